# baseline (speedup 1.0000x reference)
import functools

import jax
import jax.numpy as jnp
from jax import lax
from jax.experimental import pallas as pl
from jax.experimental.pallas import tpu as pltpu
from jax.experimental.pallas import tpu_sc as plsc

N = 10000
E = 160000
D = 256
H1 = 256
H2 = 64

CHUNK = 64
NBUF = 4
NTILES = 16
EPAD = 163840
NCHUNKS = EPAD // CHUNK
CPT = NCHUNKS // NTILES
NGRP = CPT // (2 * NBUF)


def _pack_i32(bf):
    return lax.bitcast_convert_type(
        bf.reshape(N, bf.shape[1] // 2, 2), jnp.int32)


def _mm1_body(x_ref, w_ref, oa_ref, ob_ref):
    r = jnp.dot(x_ref[...], w_ref[...], preferred_element_type=jnp.float32)
    rb = r.astype(jnp.bfloat16)
    oa_ref[...] = rb[:, : H1 // 2]
    ob_ref[...] = rb[:, H1 // 2 :]


def _matmul1(x, W1):
    TM = 1000
    return pl.pallas_call(
        _mm1_body,
        grid=(N // TM,),
        in_specs=[
            pl.BlockSpec((TM, D), lambda i: (i, 0)),
            pl.BlockSpec((D, H1), lambda i: (0, 0)),
        ],
        out_specs=[
            pl.BlockSpec((TM, H1 // 2), lambda i: (i, 0)),
            pl.BlockSpec((TM, H1 // 2), lambda i: (i, 0)),
        ],
        out_shape=[jax.ShapeDtypeStruct((N, H1 // 2), jnp.bfloat16)] * 2,
    )(x, W1)


def _mm2_body(ha_ref, hb_ref, w_ref, oa_ref, ob_ref):
    ha = jnp.maximum(ha_ref[...], 0.0)
    hb = jnp.maximum(hb_ref[...], 0.0)
    w = w_ref[...]
    r = jnp.dot(ha, w[: H1 // 2], preferred_element_type=jnp.float32)
    r = r + jnp.dot(hb, w[H1 // 2 :], preferred_element_type=jnp.float32)
    rb = r.astype(jnp.bfloat16)
    oa_ref[...] = rb[:, : H2 // 2]
    ob_ref[...] = rb[:, H2 // 2 :]


def _matmul2(h1a, h1b, W2):
    TM = 1000
    return pl.pallas_call(
        _mm2_body,
        grid=(N // TM,),
        in_specs=[
            pl.BlockSpec((TM, H1 // 2), lambda i: (i, 0)),
            pl.BlockSpec((TM, H1 // 2), lambda i: (i, 0)),
            pl.BlockSpec((H1, H2), lambda i: (0, 0)),
        ],
        out_specs=[
            pl.BlockSpec((TM, H2 // 2), lambda i: (i, 0)),
            pl.BlockSpec((TM, H2 // 2), lambda i: (i, 0)),
        ],
        out_shape=[jax.ShapeDtypeStruct((N, H2 // 2), jnp.bfloat16)] * 2,
    )(h1a, h1b, W2)


def _make_sc_aggregate(F):
    G = F // 32
    WI = F // 2
    mesh = plsc.VectorSubcoreMesh(core_axis_name="c", subcore_axis_name="s")
    NB8 = 2 * NBUF

    @functools.partial(
        pl.kernel,
        out_type=[jax.ShapeDtypeStruct((N, F), jnp.float32)] * 2,
        mesh=mesh,
        compiler_params=pltpu.CompilerParams(
            needs_layout_passes=False,
            use_tc_tiling_on_sc=False,
        ),
        scratch_types=(
            [pltpu.VMEM((3, CHUNK), jnp.int32) for _ in range(NB8)]
            + [pltpu.VMEM((CHUNK, WI), jnp.int32) for _ in range(NBUF)]
            + [pltpu.VMEM((CHUNK, F), jnp.float32) for _ in range(2)]
            + [pltpu.VMEM_SHARED((N, F), jnp.float32)]
            + [pltpu.SemaphoreType.DMA for _ in range(NB8 + NBUF + 2)]
        ),
    )
    def agg(hwa_hbm, hwb_hbm, de_hbm, zz_hbm, oa_hbm, ob_hbm, *bufs):
        de = list(bufs[0:NB8])
        rowsi = list(bufs[NB8:NB8 + NBUF])
        rowsf = list(bufs[NB8 + NBUF:NB8 + NBUF + 2])
        acc = bufs[NB8 + NBUF + 2]
        sems = list(bufs[NB8 + NBUF + 3:])
        dsem = sems[0:NB8]
        gsem = sems[NB8:NB8 + NBUF]
        ssem = sems[NB8 + NBUF:]
        c = lax.axis_index("c")
        s = lax.axis_index("s")
        base = s * CPT

        @pl.when(s == 0)
        def _():
            pltpu.sync_copy(zz_hbm, acc)

        def run(hw_hbm):
            def prefetch_de(i, b8):
                pltpu.async_copy(de_hbm.at[base + i], de[b8], dsem[b8])

            def wait_de(i, b8):
                pltpu.make_async_copy(
                    de_hbm.at[base + i], de[b8], dsem[b8]).wait()

            def gather(i, br, b8):
                pltpu.async_copy(hw_hbm.at[de[b8].at[2]], rowsi[br], gsem[br])

            def wait_gather(i, br, b8):
                pltpu.make_async_copy(
                    hw_hbm.at[de[b8].at[2]], rowsi[br], gsem[br]).wait()

            def scatter(fb, b8):
                pltpu.async_copy(rowsf[fb], acc.at[de[b8].at[0]], ssem[fb],
                                 add=True)

            def wait_scatter(fb, b8):
                pltpu.make_async_copy(
                    rowsf[fb], acc.at[de[b8].at[0]], ssem[fb]).wait()

            def convmul(br, fb, b8):
                def edge_body(j, carry):
                    ewi = plsc.load_gather(
                        de[b8], [jnp.full((16,), 1, jnp.int32),
                                 jnp.full((16,), j, jnp.int32)])
                    ewb = plsc.bitcast(ewi, jnp.float32)
                    for g in range(G):
                        vi = rowsi[br][j, pl.ds(g * 16, 16)]
                        lo = plsc.bitcast(vi << 16, jnp.float32)
                        hi = plsc.bitcast(vi & jnp.int32(-65536), jnp.float32)
                        rowsf[fb][j, pl.ds(g * 32, 16)] = lo * ewb
                        rowsf[fb][j, pl.ds(g * 32 + 16, 16)] = hi * ewb
                    return carry

                lax.fori_loop(0, CHUNK, edge_body, 0, unroll=4)

            for j in range(6):
                prefetch_de(j, j)
            for j in range(NBUF):
                wait_de(j, j)
                gather(j, j, j)
            plsc.subcore_barrier()

            def group(gg, carry):
                for b in range(NB8):
                    i = NB8 * gg + b
                    br = b % NBUF
                    fb = b % 2
                    wait_gather(i, br, b)
                    if b < 2:
                        @pl.when(gg >= 1)
                        def _():
                            wait_scatter(fb, (b - 2) % NB8)
                    else:
                        wait_scatter(fb, (b - 2) % NB8)
                    convmul(br, fb, b)
                    scatter(fb, b)
                    if b < NBUF:
                        wait_de(i + NBUF, (b + NBUF) % NB8)
                        gather(i + NBUF, br, (b + NBUF) % NB8)
                    else:
                        @pl.when(gg < NGRP - 1)
                        def _():
                            wait_de(i + NBUF, (b + NBUF) % NB8)
                            gather(i + NBUF, br, (b + NBUF) % NB8)
                    if b < 2:
                        prefetch_de(i + 6, (b + 6) % NB8)
                    else:
                        @pl.when(gg < NGRP - 1)
                        def _():
                            prefetch_de(i + 6, (b + 6) % NB8)
                return carry

            lax.fori_loop(0, NGRP, group, 0)
            wait_scatter(0, (CPT - 2) % NB8)
            wait_scatter(1, (CPT - 1) % NB8)

        @pl.when(c == 0)
        def _():
            run(hwa_hbm)

        @pl.when(c == 1)
        def _():
            run(hwb_hbm)

        plsc.subcore_barrier()

        def writeout(o_hbm):
            @pl.when(s < 15)
            def _():
                r0 = s * 624
                pltpu.sync_copy(acc.at[pl.ds(r0, 624)], o_hbm.at[pl.ds(r0, 624)])

            @pl.when(s == 15)
            def _():
                pltpu.sync_copy(acc.at[pl.ds(15 * 624, 640)],
                                o_hbm.at[pl.ds(15 * 624, 640)])

        @pl.when(c == 0)
        def _():
            writeout(oa_hbm)

        @pl.when(c == 1)
        def _():
            writeout(ob_hbm)

    return agg


_sc_agg_128 = _make_sc_aggregate(128)
_sc_agg_32 = _make_sc_aggregate(32)


def _dec_body(a0_ref, a1_ref, b0_ref, b1_ref, o_ref):
    zr = jnp.concatenate([a0_ref[...], a1_ref[...]], axis=1)
    zc = jnp.concatenate([b0_ref[...], b1_ref[...]], axis=1)
    o_ref[...] = lax.dot_general(zr, zc, (((1,), (1,)), ((), ())),
                                 preferred_element_type=jnp.float32)


def _decoder(za, zb):
    TM = 200
    G = N // TM
    return pl.pallas_call(
        _dec_body,
        grid=(G,),
        in_specs=[
            pl.BlockSpec((TM, H2 // 2), lambda i: (i, 0)),
            pl.BlockSpec((TM, H2 // 2), lambda i: (i, 0)),
            pl.BlockSpec((N, H2 // 2), lambda i: (0, 0)),
            pl.BlockSpec((N, H2 // 2), lambda i: (0, 0)),
        ],
        out_specs=pl.BlockSpec((TM, N), lambda i: (i, 0)),
        out_shape=jax.ShapeDtypeStruct((N, N), jnp.float32),
    )(za, zb, za, zb)


def _pair_perm(half_width):
    perm = []
    for g in range(half_width // 32):
        for l16 in range(16):
            perm.append(g * 32 + l16)
            perm.append(g * 32 + 16 + l16)
    return perm


_PERM_W1 = [h * 128 + p for h in range(2) for p in _pair_perm(128)]
_PERM_W2 = [h * 32 + p for h in range(2) for p in _pair_perm(32)]


def kernel(x, edge_index, edge_weight, W1, W2):
    pad = EPAD - E
    src = jnp.pad(edge_index[0], (0, pad)).reshape(NCHUNKS, 1, CHUNK)
    dst = jnp.pad(edge_index[1], (0, pad)).reshape(NCHUNKS, 1, CHUNK)
    ew = jnp.pad(edge_weight, (0, pad)).reshape(NCHUNKS, 1, CHUNK)
    de = jnp.concatenate(
        [dst, lax.bitcast_convert_type(ew, jnp.int32), src], axis=1)
    z128 = jnp.zeros((N, H1 // 2), jnp.float32)
    z32 = jnp.zeros((N, H2 // 2), jnp.float32)

    W1p = jnp.take(W1, jnp.array(_PERM_W1, jnp.int32), axis=1)
    W2p = jnp.take(W2, jnp.array(_PERM_W2, jnp.int32), axis=1)
    hw1a, hw1b = _matmul1(x, W1p)
    h1a, h1b = _sc_agg_128(_pack_i32(hw1a), _pack_i32(hw1b), de, z128)
    hw2a, hw2b = _matmul2(h1a, h1b, W2p)
    za, zb = _sc_agg_32(_pack_i32(hw2a), _pack_i32(hw2b), de, z32)
    recon = _decoder(za, zb)
    return recon.reshape(-1)


# speedup vs baseline: 1.0755x; 1.0755x over previous
import functools

import jax
import jax.numpy as jnp
from jax import lax
from jax.experimental import pallas as pl
from jax.experimental.pallas import tpu as pltpu
from jax.experimental.pallas import tpu_sc as plsc

N = 10000
E = 160000
D = 256
H1 = 256
H2 = 64

CHUNK = 64
NBUF = 4
NTILES = 16
EPAD = 163840
NCHUNKS = EPAD // CHUNK
CPT = NCHUNKS // NTILES
NGRP = CPT // (2 * NBUF)


def _pack_half(r):
    u = lax.bitcast_convert_type(r, jnp.int32)
    rne = u + 0x7FFF + jnp.bitwise_and(lax.shift_right_logical(u, 16), 1)
    bf = jnp.bitwise_and(lax.shift_right_logical(rne, 16), 0xFFFF)
    parts = []
    for g in range(r.shape[1] // 32):
        a = bf[:, g * 32 : g * 32 + 16]
        b = bf[:, g * 32 + 16 : g * 32 + 32]
        parts.append(jnp.bitwise_or(a, b << 16))
    return jnp.concatenate(parts, axis=1)


def _mm1_body(x_ref, w_ref, oa_ref, ob_ref):
    r = jnp.dot(x_ref[...], w_ref[...], preferred_element_type=jnp.float32)
    oa_ref[...] = _pack_half(r[:, : H1 // 2])
    ob_ref[...] = _pack_half(r[:, H1 // 2 :])


def _matmul1(x, W1):
    TM = 1000
    return pl.pallas_call(
        _mm1_body,
        grid=(N // TM,),
        in_specs=[
            pl.BlockSpec((TM, D), lambda i: (i, 0)),
            pl.BlockSpec((D, H1), lambda i: (0, 0)),
        ],
        out_specs=[
            pl.BlockSpec((TM, H1 // 4), lambda i: (i, 0)),
            pl.BlockSpec((TM, H1 // 4), lambda i: (i, 0)),
        ],
        out_shape=[jax.ShapeDtypeStruct((N, H1 // 4), jnp.int32)] * 2,
    )(x, W1)


def _mm2_body(ha_ref, hb_ref, w_ref, oa_ref, ob_ref):
    ha = jnp.maximum(ha_ref[...], 0.0)
    hb = jnp.maximum(hb_ref[...], 0.0)
    w = w_ref[...]
    r = jnp.dot(ha, w[: H1 // 2], preferred_element_type=jnp.float32)
    r = r + jnp.dot(hb, w[H1 // 2 :], preferred_element_type=jnp.float32)
    oa_ref[...] = _pack_half(r[:, : H2 // 2])
    ob_ref[...] = _pack_half(r[:, H2 // 2 :])


def _matmul2(h1a, h1b, W2):
    TM = 1000
    return pl.pallas_call(
        _mm2_body,
        grid=(N // TM,),
        in_specs=[
            pl.BlockSpec((TM, H1 // 2), lambda i: (i, 0)),
            pl.BlockSpec((TM, H1 // 2), lambda i: (i, 0)),
            pl.BlockSpec((H1, H2), lambda i: (0, 0)),
        ],
        out_specs=[
            pl.BlockSpec((TM, H2 // 4), lambda i: (i, 0)),
            pl.BlockSpec((TM, H2 // 4), lambda i: (i, 0)),
        ],
        out_shape=[jax.ShapeDtypeStruct((N, H2 // 4), jnp.int32)] * 2,
    )(h1a, h1b, W2)


def _make_sc_aggregate(F):
    G = F // 32
    WI = F // 2
    mesh = plsc.VectorSubcoreMesh(core_axis_name="c", subcore_axis_name="s")
    NB8 = 2 * NBUF

    @functools.partial(
        pl.kernel,
        out_type=[jax.ShapeDtypeStruct((N, F), jnp.float32)] * 2,
        mesh=mesh,
        compiler_params=pltpu.CompilerParams(
            needs_layout_passes=False,
            use_tc_tiling_on_sc=False,
        ),
        scratch_types=(
            [pltpu.VMEM((3, CHUNK), jnp.int32) for _ in range(NB8)]
            + [pltpu.VMEM((CHUNK, WI), jnp.int32) for _ in range(NBUF)]
            + [pltpu.VMEM((CHUNK, F), jnp.float32) for _ in range(2)]
            + [pltpu.VMEM_SHARED((N, F), jnp.float32)]
            + [pltpu.SemaphoreType.DMA for _ in range(NB8 + NBUF + 2)]
        ),
    )
    def agg(hwa_hbm, hwb_hbm, de_hbm, zz_hbm, oa_hbm, ob_hbm, *bufs):
        de = list(bufs[0:NB8])
        rowsi = list(bufs[NB8:NB8 + NBUF])
        rowsf = list(bufs[NB8 + NBUF:NB8 + NBUF + 2])
        acc = bufs[NB8 + NBUF + 2]
        sems = list(bufs[NB8 + NBUF + 3:])
        dsem = sems[0:NB8]
        gsem = sems[NB8:NB8 + NBUF]
        ssem = sems[NB8 + NBUF:]
        c = lax.axis_index("c")
        s = lax.axis_index("s")
        base = s * CPT

        @pl.when(s == 0)
        def _():
            pltpu.sync_copy(zz_hbm, acc)

        def run(hw_hbm):
            def prefetch_de(i, b8):
                pltpu.async_copy(de_hbm.at[base + i], de[b8], dsem[b8])

            def wait_de(i, b8):
                pltpu.make_async_copy(
                    de_hbm.at[base + i], de[b8], dsem[b8]).wait()

            def gather(i, br, b8):
                pltpu.async_copy(hw_hbm.at[de[b8].at[2]], rowsi[br], gsem[br])

            def wait_gather(i, br, b8):
                pltpu.make_async_copy(
                    hw_hbm.at[de[b8].at[2]], rowsi[br], gsem[br]).wait()

            def scatter(fb, b8):
                pltpu.async_copy(rowsf[fb], acc.at[de[b8].at[0]], ssem[fb],
                                 add=True)

            def wait_scatter(fb, b8):
                pltpu.make_async_copy(
                    rowsf[fb], acc.at[de[b8].at[0]], ssem[fb]).wait()

            def convmul(br, fb, b8):
                def edge_body(j, carry):
                    ewi = plsc.load_gather(
                        de[b8], [jnp.full((16,), 1, jnp.int32),
                                 jnp.full((16,), j, jnp.int32)])
                    ewb = plsc.bitcast(ewi, jnp.float32)
                    for g in range(G):
                        vi = rowsi[br][j, pl.ds(g * 16, 16)]
                        lo = plsc.bitcast(vi << 16, jnp.float32)
                        hi = plsc.bitcast(vi & jnp.int32(-65536), jnp.float32)
                        rowsf[fb][j, pl.ds(g * 32, 16)] = lo * ewb
                        rowsf[fb][j, pl.ds(g * 32 + 16, 16)] = hi * ewb
                    return carry

                lax.fori_loop(0, CHUNK, edge_body, 0, unroll=4)

            for j in range(6):
                prefetch_de(j, j)
            for j in range(NBUF):
                wait_de(j, j)
                gather(j, j, j)
            plsc.subcore_barrier()

            def group(gg, carry):
                for b in range(NB8):
                    i = NB8 * gg + b
                    br = b % NBUF
                    fb = b % 2
                    wait_gather(i, br, b)
                    if b < 2:
                        @pl.when(gg >= 1)
                        def _():
                            wait_scatter(fb, (b - 2) % NB8)
                    else:
                        wait_scatter(fb, (b - 2) % NB8)
                    convmul(br, fb, b)
                    scatter(fb, b)
                    if b < NBUF:
                        wait_de(i + NBUF, (b + NBUF) % NB8)
                        gather(i + NBUF, br, (b + NBUF) % NB8)
                    else:
                        @pl.when(gg < NGRP - 1)
                        def _():
                            wait_de(i + NBUF, (b + NBUF) % NB8)
                            gather(i + NBUF, br, (b + NBUF) % NB8)
                    if b < 2:
                        prefetch_de(i + 6, (b + 6) % NB8)
                    else:
                        @pl.when(gg < NGRP - 1)
                        def _():
                            prefetch_de(i + 6, (b + 6) % NB8)
                return carry

            lax.fori_loop(0, NGRP, group, 0)
            wait_scatter(0, (CPT - 2) % NB8)
            wait_scatter(1, (CPT - 1) % NB8)

        @pl.when(c == 0)
        def _():
            run(hwa_hbm)

        @pl.when(c == 1)
        def _():
            run(hwb_hbm)

        plsc.subcore_barrier()

        def writeout(o_hbm):
            @pl.when(s < 15)
            def _():
                r0 = s * 624
                pltpu.sync_copy(acc.at[pl.ds(r0, 624)], o_hbm.at[pl.ds(r0, 624)])

            @pl.when(s == 15)
            def _():
                pltpu.sync_copy(acc.at[pl.ds(15 * 624, 640)],
                                o_hbm.at[pl.ds(15 * 624, 640)])

        @pl.when(c == 0)
        def _():
            writeout(oa_hbm)

        @pl.when(c == 1)
        def _():
            writeout(ob_hbm)

    return agg


_sc_agg_128 = _make_sc_aggregate(128)
_sc_agg_32 = _make_sc_aggregate(32)


def _dec_body(a0_ref, a1_ref, b0_ref, b1_ref, o_ref):
    zr = jnp.concatenate([a0_ref[...], a1_ref[...]], axis=1)
    zc = jnp.concatenate([b0_ref[...], b1_ref[...]], axis=1)
    o_ref[...] = lax.dot_general(zr, zc, (((1,), (1,)), ((), ())),
                                 preferred_element_type=jnp.float32)


def _decoder(za, zb):
    TM = 200
    G = N // TM
    return pl.pallas_call(
        _dec_body,
        grid=(G,),
        in_specs=[
            pl.BlockSpec((TM, H2 // 2), lambda i: (i, 0)),
            pl.BlockSpec((TM, H2 // 2), lambda i: (i, 0)),
            pl.BlockSpec((N, H2 // 2), lambda i: (0, 0)),
            pl.BlockSpec((N, H2 // 2), lambda i: (0, 0)),
        ],
        out_specs=pl.BlockSpec((TM, N), lambda i: (i, 0)),
        out_shape=jax.ShapeDtypeStruct((N, N), jnp.float32),
    )(za, zb, za, zb)


def kernel(x, edge_index, edge_weight, W1, W2):
    pad = EPAD - E
    src = jnp.pad(edge_index[0], (0, pad)).reshape(NCHUNKS, 1, CHUNK)
    dst = jnp.pad(edge_index[1], (0, pad)).reshape(NCHUNKS, 1, CHUNK)
    ew = jnp.pad(edge_weight, (0, pad)).reshape(NCHUNKS, 1, CHUNK)
    de = jnp.concatenate(
        [dst, lax.bitcast_convert_type(ew, jnp.int32), src], axis=1)
    z128 = jnp.zeros((N, H1 // 2), jnp.float32)
    z32 = jnp.zeros((N, H2 // 2), jnp.float32)

    hw1a, hw1b = _matmul1(x, W1)
    h1a, h1b = _sc_agg_128(hw1a, hw1b, de, z128)
    hw2a, hw2b = _matmul2(h1a, h1b, W2)
    za, zb = _sc_agg_32(hw2a, hw2b, de, z32)
    recon = _decoder(za, zb)
    return recon.reshape(-1)


# speedup vs baseline: 1.0764x; 1.0009x over previous
import functools

import jax
import jax.numpy as jnp
from jax import lax
from jax.experimental import pallas as pl
from jax.experimental.pallas import tpu as pltpu
from jax.experimental.pallas import tpu_sc as plsc

N = 10000
E = 160000
D = 256
H1 = 256
H2 = 64

CHUNK = 64
NBUF = 4
NTILES = 16
EPAD = 163840
NCHUNKS = EPAD // CHUNK
CPT = NCHUNKS // NTILES
NGRP = CPT // (2 * NBUF)


def _pack_half(r):
    u = lax.bitcast_convert_type(r, jnp.int32)
    rne = u + 0x7FFF + jnp.bitwise_and(lax.shift_right_logical(u, 16), 1)
    bf = jnp.bitwise_and(lax.shift_right_logical(rne, 16), 0xFFFF)
    parts = []
    for g in range(r.shape[1] // 32):
        a = bf[:, g * 32 : g * 32 + 16]
        b = bf[:, g * 32 + 16 : g * 32 + 32]
        parts.append(jnp.bitwise_or(a, b << 16))
    return jnp.concatenate(parts, axis=1)


def _mm1_body(x_ref, w_ref, oa_ref, ob_ref):
    r = jnp.dot(x_ref[...], w_ref[...], preferred_element_type=jnp.float32)
    oa_ref[...] = _pack_half(r[:, : H1 // 2])
    ob_ref[...] = _pack_half(r[:, H1 // 2 :])


def _matmul1(x, W1):
    TM = 1000
    return pl.pallas_call(
        _mm1_body,
        grid=(N // TM,),
        in_specs=[
            pl.BlockSpec((TM, D), lambda i: (i, 0)),
            pl.BlockSpec((D, H1), lambda i: (0, 0)),
        ],
        out_specs=[
            pl.BlockSpec((TM, H1 // 4), lambda i: (i, 0)),
            pl.BlockSpec((TM, H1 // 4), lambda i: (i, 0)),
        ],
        out_shape=[jax.ShapeDtypeStruct((N, H1 // 4), jnp.int32)] * 2,
    )(x, W1)


def _mm2_body(ha_ref, hb_ref, w_ref, oa_ref, ob_ref):
    ha = jnp.maximum(ha_ref[...], 0.0)
    hb = jnp.maximum(hb_ref[...], 0.0)
    w = w_ref[...]
    r = jnp.dot(ha, w[: H1 // 2], preferred_element_type=jnp.float32)
    r = r + jnp.dot(hb, w[H1 // 2 :], preferred_element_type=jnp.float32)
    oa_ref[...] = _pack_half(r[:, : H2 // 2])
    ob_ref[...] = _pack_half(r[:, H2 // 2 :])


def _matmul2(h1a, h1b, W2):
    TM = 1000
    return pl.pallas_call(
        _mm2_body,
        grid=(N // TM,),
        in_specs=[
            pl.BlockSpec((TM, H1 // 2), lambda i: (i, 0)),
            pl.BlockSpec((TM, H1 // 2), lambda i: (i, 0)),
            pl.BlockSpec((H1, H2), lambda i: (0, 0)),
        ],
        out_specs=[
            pl.BlockSpec((TM, H2 // 4), lambda i: (i, 0)),
            pl.BlockSpec((TM, H2 // 4), lambda i: (i, 0)),
        ],
        out_shape=[jax.ShapeDtypeStruct((N, H2 // 4), jnp.int32)] * 2,
    )(h1a, h1b, W2)


def _make_sc_aggregate(F):
    G = F // 32
    WI = F // 2
    mesh = plsc.VectorSubcoreMesh(core_axis_name="c", subcore_axis_name="s")
    NB8 = 2 * NBUF

    @functools.partial(
        pl.kernel,
        out_type=[jax.ShapeDtypeStruct((N, F), jnp.float32)] * 2,
        mesh=mesh,
        compiler_params=pltpu.CompilerParams(
            needs_layout_passes=False,
            use_tc_tiling_on_sc=False,
        ),
        scratch_types=(
            [pltpu.VMEM((3, CHUNK), jnp.int32) for _ in range(NB8)]
            + [pltpu.VMEM((CHUNK, WI), jnp.int32) for _ in range(NBUF)]
            + [pltpu.VMEM((CHUNK, F), jnp.float32) for _ in range(2)]
            + [pltpu.VMEM_SHARED((N, F), jnp.float32)]
            + [pltpu.SemaphoreType.DMA for _ in range(NB8 + NBUF + 2)]
        ),
    )
    def agg(hwa_hbm, hwb_hbm, de_hbm, zz_hbm, oa_hbm, ob_hbm, *bufs):
        de = list(bufs[0:NB8])
        rowsi = list(bufs[NB8:NB8 + NBUF])
        rowsf = list(bufs[NB8 + NBUF:NB8 + NBUF + 2])
        acc = bufs[NB8 + NBUF + 2]
        sems = list(bufs[NB8 + NBUF + 3:])
        dsem = sems[0:NB8]
        gsem = sems[NB8:NB8 + NBUF]
        ssem = sems[NB8 + NBUF:]
        c = lax.axis_index("c")
        s = lax.axis_index("s")
        base = s * CPT

        @pl.when(s == 0)
        def _():
            pltpu.sync_copy(zz_hbm, acc)

        def run(hw_hbm):
            def prefetch_de(i, b8):
                pltpu.async_copy(de_hbm.at[base + i], de[b8], dsem[b8])

            def wait_de(i, b8):
                pltpu.make_async_copy(
                    de_hbm.at[base + i], de[b8], dsem[b8]).wait()

            def gather(i, br, b8):
                pltpu.async_copy(hw_hbm.at[de[b8].at[2]], rowsi[br], gsem[br])

            def wait_gather(i, br, b8):
                pltpu.make_async_copy(
                    hw_hbm.at[de[b8].at[2]], rowsi[br], gsem[br]).wait()

            def scatter(fb, b8):
                pltpu.async_copy(rowsf[fb], acc.at[de[b8].at[0]], ssem[fb],
                                 add=True)

            def wait_scatter(fb, b8):
                pltpu.make_async_copy(
                    rowsf[fb], acc.at[de[b8].at[0]], ssem[fb]).wait()

            def convmul(br, fb, b8):
                def edge_body(j, carry):
                    ewi = plsc.load_gather(
                        de[b8], [jnp.full((16,), 1, jnp.int32),
                                 jnp.full((16,), j, jnp.int32)])
                    ewb = plsc.bitcast(ewi, jnp.float32)
                    for g in range(G):
                        vi = rowsi[br][j, pl.ds(g * 16, 16)]
                        lo = plsc.bitcast(vi << 16, jnp.float32)
                        hi = plsc.bitcast(vi & jnp.int32(-65536), jnp.float32)
                        rowsf[fb][j, pl.ds(g * 32, 16)] = lo * ewb
                        rowsf[fb][j, pl.ds(g * 32 + 16, 16)] = hi * ewb
                    return carry

                lax.fori_loop(0, CHUNK, edge_body, 0, unroll=4)

            for j in range(6):
                prefetch_de(j, j)
            for j in range(NBUF):
                wait_de(j, j)
                gather(j, j, j)
            plsc.subcore_barrier()

            def group(gg, carry):
                for b in range(NB8):
                    i = NB8 * gg + b
                    br = b % NBUF
                    fb = b % 2
                    wait_gather(i, br, b)
                    if b < 2:
                        @pl.when(gg >= 1)
                        def _():
                            wait_scatter(fb, (b - 2) % NB8)
                    else:
                        wait_scatter(fb, (b - 2) % NB8)
                    convmul(br, fb, b)
                    scatter(fb, b)
                    if b < NBUF:
                        wait_de(i + NBUF, (b + NBUF) % NB8)
                        gather(i + NBUF, br, (b + NBUF) % NB8)
                    else:
                        @pl.when(gg < NGRP - 1)
                        def _():
                            wait_de(i + NBUF, (b + NBUF) % NB8)
                            gather(i + NBUF, br, (b + NBUF) % NB8)
                    if b < 2:
                        prefetch_de(i + 6, (b + 6) % NB8)
                    else:
                        @pl.when(gg < NGRP - 1)
                        def _():
                            prefetch_de(i + 6, (b + 6) % NB8)
                return carry

            lax.fori_loop(0, NGRP, group, 0)
            wait_scatter(0, (CPT - 2) % NB8)
            wait_scatter(1, (CPT - 1) % NB8)

        @pl.when(c == 0)
        def _():
            run(hwa_hbm)

        @pl.when(c == 1)
        def _():
            run(hwb_hbm)

        plsc.subcore_barrier()

        def writeout(o_hbm):
            @pl.when(s < 15)
            def _():
                r0 = s * 624
                pltpu.sync_copy(acc.at[pl.ds(r0, 624)], o_hbm.at[pl.ds(r0, 624)])

            @pl.when(s == 15)
            def _():
                pltpu.sync_copy(acc.at[pl.ds(15 * 624, 640)],
                                o_hbm.at[pl.ds(15 * 624, 640)])

        @pl.when(c == 0)
        def _():
            writeout(oa_hbm)

        @pl.when(c == 1)
        def _():
            writeout(ob_hbm)

    return agg


_sc_agg_128 = _make_sc_aggregate(128)
_sc_agg_32 = _make_sc_aggregate(32)


def _dec_body(a0_ref, a1_ref, b0_ref, b1_ref, o_ref):
    zr = jnp.concatenate([a0_ref[...], a1_ref[...]], axis=1)
    zc = jnp.concatenate([b0_ref[...], b1_ref[...]], axis=1)
    o_ref[...] = lax.dot_general(zr, zc, (((1,), (1,)), ((), ())),
                                 preferred_element_type=jnp.float32)


def _decoder(za, zb):
    TM = 400
    G = N // TM
    return pl.pallas_call(
        _dec_body,
        grid=(G,),
        in_specs=[
            pl.BlockSpec((TM, H2 // 2), lambda i: (i, 0)),
            pl.BlockSpec((TM, H2 // 2), lambda i: (i, 0)),
            pl.BlockSpec((N, H2 // 2), lambda i: (0, 0)),
            pl.BlockSpec((N, H2 // 2), lambda i: (0, 0)),
        ],
        out_specs=pl.BlockSpec((TM, N), lambda i: (i, 0)),
        out_shape=jax.ShapeDtypeStruct((N, N), jnp.float32),
    )(za, zb, za, zb)


def kernel(x, edge_index, edge_weight, W1, W2):
    pad = EPAD - E
    src = jnp.pad(edge_index[0], (0, pad)).reshape(NCHUNKS, 1, CHUNK)
    dst = jnp.pad(edge_index[1], (0, pad)).reshape(NCHUNKS, 1, CHUNK)
    ew = jnp.pad(edge_weight, (0, pad)).reshape(NCHUNKS, 1, CHUNK)
    de = jnp.concatenate(
        [dst, lax.bitcast_convert_type(ew, jnp.int32), src], axis=1)
    z128 = jnp.zeros((N, H1 // 2), jnp.float32)
    z32 = jnp.zeros((N, H2 // 2), jnp.float32)

    hw1a, hw1b = _matmul1(x, W1)
    h1a, h1b = _sc_agg_128(hw1a, hw1b, de, z128)
    hw2a, hw2b = _matmul2(h1a, h1b, W2)
    za, zb = _sc_agg_32(hw2a, hw2b, de, z32)
    recon = _decoder(za, zb)
    return recon.reshape(-1)


# speedup vs baseline: 1.0860x; 1.0089x over previous
import functools

import jax
import jax.numpy as jnp
from jax import lax
from jax.experimental import pallas as pl
from jax.experimental.pallas import tpu as pltpu
from jax.experimental.pallas import tpu_sc as plsc

N = 10000
E = 160000
D = 256
H1 = 256
H2 = 64

CHUNK = 64
NBUF = 4
NTILES = 16
EPAD = 163840
NCHUNKS = EPAD // CHUNK
CPT = NCHUNKS // NTILES
NGRP = CPT // (2 * NBUF)


def _pack_half(r):
    u = lax.bitcast_convert_type(r, jnp.int32)
    rne = u + 0x7FFF + jnp.bitwise_and(lax.shift_right_logical(u, 16), 1)
    bf = jnp.bitwise_and(lax.shift_right_logical(rne, 16), 0xFFFF)
    parts = []
    for g in range(r.shape[1] // 32):
        a = bf[:, g * 32 : g * 32 + 16]
        b = bf[:, g * 32 + 16 : g * 32 + 32]
        parts.append(jnp.bitwise_or(a, b << 16))
    return jnp.concatenate(parts, axis=1)


def _mm1_body(x_ref, w_ref, oa_ref, ob_ref):
    r = jnp.dot(x_ref[...], w_ref[...], preferred_element_type=jnp.float32)
    oa_ref[...] = _pack_half(r[:, : H1 // 2])
    ob_ref[...] = _pack_half(r[:, H1 // 2 :])


def _matmul1(x, W1):
    TM = 1000
    return pl.pallas_call(
        _mm1_body,
        grid=(N // TM,),
        in_specs=[
            pl.BlockSpec((TM, D), lambda i: (i, 0)),
            pl.BlockSpec((D, H1), lambda i: (0, 0)),
        ],
        out_specs=[
            pl.BlockSpec((TM, H1 // 4), lambda i: (i, 0)),
            pl.BlockSpec((TM, H1 // 4), lambda i: (i, 0)),
        ],
        out_shape=[jax.ShapeDtypeStruct((N, H1 // 4), jnp.int32)] * 2,
    )(x, W1)


def _mm2_body(ha_ref, hb_ref, w_ref, oa_ref, ob_ref):
    ha = jnp.maximum(ha_ref[...], 0.0)
    hb = jnp.maximum(hb_ref[...], 0.0)
    w = w_ref[...]
    r = jnp.dot(ha, w[: H1 // 2], preferred_element_type=jnp.float32)
    r = r + jnp.dot(hb, w[H1 // 2 :], preferred_element_type=jnp.float32)
    oa_ref[...] = _pack_half(r[:, : H2 // 2])
    ob_ref[...] = _pack_half(r[:, H2 // 2 :])


def _matmul2(h1a, h1b, W2):
    TM = 1000
    return pl.pallas_call(
        _mm2_body,
        grid=(N // TM,),
        in_specs=[
            pl.BlockSpec((TM, H1 // 2), lambda i: (i, 0)),
            pl.BlockSpec((TM, H1 // 2), lambda i: (i, 0)),
            pl.BlockSpec((H1, H2), lambda i: (0, 0)),
        ],
        out_specs=[
            pl.BlockSpec((TM, H2 // 4), lambda i: (i, 0)),
            pl.BlockSpec((TM, H2 // 4), lambda i: (i, 0)),
        ],
        out_shape=[jax.ShapeDtypeStruct((N, H2 // 4), jnp.int32)] * 2,
    )(h1a, h1b, W2)


def _make_sc_aggregate(F):
    G = F // 32
    WI = F // 2
    mesh = plsc.VectorSubcoreMesh(core_axis_name="c", subcore_axis_name="s")
    NB8 = 2 * NBUF

    @functools.partial(
        pl.kernel,
        out_type=[jax.ShapeDtypeStruct((N, F), jnp.float32)] * 2,
        mesh=mesh,
        compiler_params=pltpu.CompilerParams(
            needs_layout_passes=False,
            use_tc_tiling_on_sc=False,
        ),
        scratch_types=(
            [pltpu.VMEM((3, CHUNK), jnp.int32) for _ in range(NB8)]
            + [pltpu.VMEM((CHUNK, WI), jnp.int32) for _ in range(NBUF)]
            + [pltpu.VMEM((CHUNK, F), jnp.float32) for _ in range(2)]
            + [pltpu.VMEM_SHARED((N, F), jnp.float32)]
            + [pltpu.SemaphoreType.DMA for _ in range(NB8 + NBUF + 2)]
        ),
    )
    def agg(hwa_hbm, hwb_hbm, de_hbm, oa_hbm, ob_hbm, *bufs):
        de = list(bufs[0:NB8])
        rowsi = list(bufs[NB8:NB8 + NBUF])
        rowsf = list(bufs[NB8 + NBUF:NB8 + NBUF + 2])
        acc = bufs[NB8 + NBUF + 2]
        sems = list(bufs[NB8 + NBUF + 3:])
        dsem = sems[0:NB8]
        gsem = sems[NB8:NB8 + NBUF]
        ssem = sems[NB8 + NBUF:]
        c = lax.axis_index("c")
        s = lax.axis_index("s")
        base = s * CPT

        def zrow(j, carry):
            for kk in range(F // 16):
                rowsf[0][j, pl.ds(kk * 16, 16)] = jnp.zeros((16,), jnp.float32)
            return carry

        lax.fori_loop(0, CHUNK, zrow, 0, unroll=4)
        r0 = s * 624

        @pl.when(s < 15)
        def _():
            for k in range(9):
                pltpu.sync_copy(rowsf[0], acc.at[pl.ds(r0 + k * 64, 64)])
            pltpu.sync_copy(rowsf[0].at[pl.ds(0, 48)],
                            acc.at[pl.ds(r0 + 576, 48)])

        @pl.when(s == 15)
        def _():
            for k in range(10):
                pltpu.sync_copy(rowsf[0], acc.at[pl.ds(15 * 624 + k * 64, 64)])

        def run(hw_hbm):
            def prefetch_de(i, b8):
                pltpu.async_copy(de_hbm.at[base + i], de[b8], dsem[b8])

            def wait_de(i, b8):
                pltpu.make_async_copy(
                    de_hbm.at[base + i], de[b8], dsem[b8]).wait()

            def gather(i, br, b8):
                pltpu.async_copy(hw_hbm.at[de[b8].at[2]], rowsi[br], gsem[br])

            def wait_gather(i, br, b8):
                pltpu.make_async_copy(
                    hw_hbm.at[de[b8].at[2]], rowsi[br], gsem[br]).wait()

            def scatter(fb, b8):
                pltpu.async_copy(rowsf[fb], acc.at[de[b8].at[0]], ssem[fb],
                                 add=True)

            def wait_scatter(fb, b8):
                pltpu.make_async_copy(
                    rowsf[fb], acc.at[de[b8].at[0]], ssem[fb]).wait()

            def convmul(br, fb, b8):
                def edge_body(j, carry):
                    ewi = plsc.load_gather(
                        de[b8], [jnp.full((16,), 1, jnp.int32),
                                 jnp.full((16,), j, jnp.int32)])
                    ewb = plsc.bitcast(ewi, jnp.float32)
                    for g in range(G):
                        vi = rowsi[br][j, pl.ds(g * 16, 16)]
                        lo = plsc.bitcast(vi << 16, jnp.float32)
                        hi = plsc.bitcast(vi & jnp.int32(-65536), jnp.float32)
                        rowsf[fb][j, pl.ds(g * 32, 16)] = lo * ewb
                        rowsf[fb][j, pl.ds(g * 32 + 16, 16)] = hi * ewb
                    return carry

                lax.fori_loop(0, CHUNK, edge_body, 0, unroll=4)

            for j in range(6):
                prefetch_de(j, j)
            for j in range(NBUF):
                wait_de(j, j)
                gather(j, j, j)
            plsc.subcore_barrier()

            def group(gg, carry):
                for b in range(NB8):
                    i = NB8 * gg + b
                    br = b % NBUF
                    fb = b % 2
                    wait_gather(i, br, b)
                    if b < 2:
                        @pl.when(gg >= 1)
                        def _():
                            wait_scatter(fb, (b - 2) % NB8)
                    else:
                        wait_scatter(fb, (b - 2) % NB8)
                    convmul(br, fb, b)
                    scatter(fb, b)
                    if b < NBUF:
                        wait_de(i + NBUF, (b + NBUF) % NB8)
                        gather(i + NBUF, br, (b + NBUF) % NB8)
                    else:
                        @pl.when(gg < NGRP - 1)
                        def _():
                            wait_de(i + NBUF, (b + NBUF) % NB8)
                            gather(i + NBUF, br, (b + NBUF) % NB8)
                    if b < 2:
                        prefetch_de(i + 6, (b + 6) % NB8)
                    else:
                        @pl.when(gg < NGRP - 1)
                        def _():
                            prefetch_de(i + 6, (b + 6) % NB8)
                return carry

            lax.fori_loop(0, NGRP, group, 0)
            wait_scatter(0, (CPT - 2) % NB8)
            wait_scatter(1, (CPT - 1) % NB8)

        @pl.when(c == 0)
        def _():
            run(hwa_hbm)

        @pl.when(c == 1)
        def _():
            run(hwb_hbm)

        plsc.subcore_barrier()

        def writeout(o_hbm):
            @pl.when(s < 15)
            def _():
                r0 = s * 624
                pltpu.sync_copy(acc.at[pl.ds(r0, 624)], o_hbm.at[pl.ds(r0, 624)])

            @pl.when(s == 15)
            def _():
                pltpu.sync_copy(acc.at[pl.ds(15 * 624, 640)],
                                o_hbm.at[pl.ds(15 * 624, 640)])

        @pl.when(c == 0)
        def _():
            writeout(oa_hbm)

        @pl.when(c == 1)
        def _():
            writeout(ob_hbm)

    return agg


_sc_agg_128 = _make_sc_aggregate(128)
_sc_agg_32 = _make_sc_aggregate(32)


def _dec_body(a0_ref, a1_ref, b0_ref, b1_ref, o_ref):
    zr = jnp.concatenate([a0_ref[...], a1_ref[...]], axis=1)
    zc = jnp.concatenate([b0_ref[...], b1_ref[...]], axis=1)
    o_ref[...] = lax.dot_general(zr, zc, (((1,), (1,)), ((), ())),
                                 preferred_element_type=jnp.float32)


def _decoder(za, zb):
    TM = 400
    G = N // TM
    return pl.pallas_call(
        _dec_body,
        grid=(G,),
        in_specs=[
            pl.BlockSpec((TM, H2 // 2), lambda i: (i, 0)),
            pl.BlockSpec((TM, H2 // 2), lambda i: (i, 0)),
            pl.BlockSpec((N, H2 // 2), lambda i: (0, 0)),
            pl.BlockSpec((N, H2 // 2), lambda i: (0, 0)),
        ],
        out_specs=pl.BlockSpec((TM, N), lambda i: (i, 0)),
        out_shape=jax.ShapeDtypeStruct((N, N), jnp.float32),
    )(za, zb, za, zb)


def kernel(x, edge_index, edge_weight, W1, W2):
    pad = EPAD - E
    src = jnp.pad(edge_index[0], (0, pad)).reshape(NCHUNKS, 1, CHUNK)
    dst = jnp.pad(edge_index[1], (0, pad)).reshape(NCHUNKS, 1, CHUNK)
    ew = jnp.pad(edge_weight, (0, pad)).reshape(NCHUNKS, 1, CHUNK)
    de = jnp.concatenate(
        [dst, lax.bitcast_convert_type(ew, jnp.int32), src], axis=1)

    hw1a, hw1b = _matmul1(x, W1)
    h1a, h1b = _sc_agg_128(hw1a, hw1b, de)
    hw2a, hw2b = _matmul2(h1a, h1b, W2)
    za, zb = _sc_agg_32(hw2a, hw2b, de)
    recon = _decoder(za, zb)
    return recon.reshape(-1)
